# Initial kernel scaffold; baseline (speedup 1.0000x reference)
import jax
import jax.numpy as jnp
from jax.experimental import pallas as pl


def kernel(input, weight):
    raise NotImplementedError("write your pallas kernel here")


# speedup vs baseline: 1.4951x; 1.4951x over previous
import functools

import jax
import jax.numpy as jnp
from jax import lax
from jax.experimental import pallas as pl
from jax.experimental.pallas import tpu as pltpu
from jax.experimental.pallas import tpu_sc as plsc

D_EMB = 32
NUM_WORKERS = 32


def _embed_lookup(idx, weight, total, chunk):
    b_per_w = total // NUM_WORKERS
    n_chunks = b_per_w // chunk
    mesh = plsc.VectorSubcoreMesh(core_axis_name="c", subcore_axis_name="s")

    @functools.partial(
        pl.kernel,
        mesh=mesh,
        out_type=jax.ShapeDtypeStruct((total, D_EMB), jnp.float32),
        compiler_params=pltpu.CompilerParams(use_tc_tiling_on_sc=False),
        scratch_types=[
            pltpu.VMEM((chunk,), jnp.int32),
            pltpu.VMEM((chunk, D_EMB), jnp.float32),
            pltpu.SemaphoreType.DMA,
        ],
    )
    def k(idx_hbm, table_hbm, out_hbm, idx_v, rows_v, sem):
        wid = lax.axis_index("s") * 2 + lax.axis_index("c")
        base = wid * b_per_w

        def body(j, carry):
            off = base + j * chunk
            pltpu.sync_copy(idx_hbm.at[pl.ds(off, chunk)], idx_v)
            pltpu.async_copy(table_hbm.at[idx_v], rows_v, sem).wait()
            pltpu.sync_copy(rows_v, out_hbm.at[pl.ds(off, chunk)])
            return carry

        lax.fori_loop(0, n_chunks, body, 0)

    return k(idx, weight)


def kernel(input, weight):
    b, s = input.shape
    total = b * s
    idx = input.reshape(total).astype(jnp.int32)
    out = _embed_lookup(idx, weight, total, chunk=3200)
    return out.reshape(b, s, D_EMB)


# speedup vs baseline: 1.4960x; 1.0006x over previous
import functools

import jax
import jax.numpy as jnp
from jax import lax
from jax.experimental import pallas as pl
from jax.experimental.pallas import tpu as pltpu
from jax.experimental.pallas import tpu_sc as plsc

D_EMB = 32
NUM_WORKERS = 32
NBUF = 4
CHUNK = 800


def _embed_lookup(idx, weight, total):
    b_per_w = total // NUM_WORKERS
    n_chunks = b_per_w // CHUNK
    n_rounds = n_chunks // NBUF
    assert n_chunks % NBUF == 0 and n_rounds >= 2
    mesh = plsc.VectorSubcoreMesh(core_axis_name="c", subcore_axis_name="s")

    @functools.partial(
        pl.kernel,
        mesh=mesh,
        out_type=jax.ShapeDtypeStruct((total, D_EMB), jnp.float32),
        compiler_params=pltpu.CompilerParams(use_tc_tiling_on_sc=False),
        scratch_types=(
            [pltpu.VMEM((b_per_w,), jnp.int32)]
            + [pltpu.VMEM((CHUNK, D_EMB), jnp.float32) for _ in range(NBUF)]
            + [pltpu.SemaphoreType.DMA for _ in range(2 * NBUF)]
        ),
    )
    def k(idx_hbm, table_hbm, out_hbm, idx_v, *bufs_and_sems):
        rows = bufs_and_sems[:NBUF]
        sem_g = bufs_and_sems[NBUF:2 * NBUF]
        sem_s = bufs_and_sems[2 * NBUF:]
        wid = lax.axis_index("s") * 2 + lax.axis_index("c")
        base = wid * b_per_w

        pltpu.sync_copy(idx_hbm.at[pl.ds(base, b_per_w)], idx_v)

        def wait(sem, buf):
            pltpu.make_async_copy(out_hbm.at[pl.ds(0, CHUNK)], buf, sem).wait()

        def gather(j, b):
            off = pl.multiple_of(j * CHUNK, CHUNK)
            pltpu.async_copy(
                table_hbm.at[idx_v.at[pl.ds(off, CHUNK)]], rows[b], sem_g[b])

        def store(j, b):
            off = pl.multiple_of(base + j * CHUNK, CHUNK)
            pltpu.async_copy(rows[b], out_hbm.at[pl.ds(off, CHUNK)], sem_s[b])

        for b in range(NBUF):
            gather(b, b)
            if b > 0:
                wait(sem_g[b - 1], rows[b - 1])
                store(b - 1, b - 1)

        def round_body(r, carry):
            for b in range(NBUF):
                j = r * NBUF + b
                wait(sem_s[b], rows[b])
                gather(j, b)
                pb = (b - 1) % NBUF
                wait(sem_g[pb], rows[pb])
                store(j - 1, pb)
            return carry

        lax.fori_loop(1, n_rounds, round_body, 0)

        last_b = NBUF - 1
        wait(sem_g[last_b], rows[last_b])
        store(n_chunks - 1, last_b)
        for b in range(NBUF):
            wait(sem_s[b], rows[b])

    return k(idx, weight)


def kernel(input, weight):
    b, s = input.shape
    total = b * s
    idx = input.reshape(total).astype(jnp.int32)
    out = _embed_lookup(idx, weight, total)
    return out.reshape(b, s, D_EMB)
